# baseline (speedup 1.0000x reference)
import functools

import jax
import jax.numpy as jnp
from jax import lax
from jax.experimental import pallas as pl
from jax.experimental.pallas import tpu as pltpu
from jax.experimental.pallas import tpu_sc as plsc

_NUM_EMB = 1000000
_D = 64
_B, _S, _P = 1024, 50, 4
_K = _S * _P
_TOTAL = _B * _K
_NW = 32
_BPW = _B // _NW
_C0, _C1 = 128, 72
_BB = 16


def _prep_body(pids_ref, flat_ref, rw_ref):
    row = lax.broadcasted_iota(jnp.int32, (_S, _K), 0)
    col = lax.broadcasted_iota(jnp.int32, (_S, _K), 1)
    bigmask = ((col >> 2) == row).astype(jnp.float32)
    pr = lax.broadcasted_iota(jnp.int32, (_P, _K), 0)
    pc = lax.broadcasted_iota(jnp.int32, (_P, _K), 1)
    gmatf = (pr == (pc & 3)).astype(jnp.float32)
    onescol = jnp.ones((_S, 1), jnp.float32)
    kr = lax.broadcasted_iota(jnp.int32, (_K, _K), 0)
    kc = lax.broadcasted_iota(jnp.int32, (_K, _K), 1)
    identf = (kr == kc).astype(jnp.float32)
    for i in range(_BB):
        ids3 = pids_ref[i]
        comps = [
            (ids3 & 127).astype(jnp.float32),
            ((ids3 >> 7) & 127).astype(jnp.float32),
            (ids3 >> 14).astype(jnp.float32),
        ]
        rows = []
        cols = []
        for c in comps:
            y = lax.dot_general(
                c, gmatf, (((1,), (0,)), ((), ())),
                preferred_element_type=jnp.float32,
            )
            z = y * bigmask
            rc = lax.dot_general(
                onescol, z, (((0,), (0,)), ((), ())),
                preferred_element_type=jnp.float32,
            )
            cc = lax.dot_general(
                identf, rc, (((1,), (1,)), ((), ())),
                preferred_element_type=jnp.float32,
            )
            rows.append(rc)
            cols.append(cc)
        ids_row = rows[0] + 128.0 * rows[1] + 16384.0 * rows[2]
        ids_col = cols[0] + 128.0 * cols[1] + 16384.0 * cols[2]
        eqf = (ids_col == ids_row).astype(jnp.float32)
        cnt = jnp.sum(eqf, axis=0, keepdims=True)
        flat_ref[i : i + 1] = ids_row.astype(jnp.int32)
        rw_ref[i : i + 1] = 1.0 / cnt


def _prep(pids, interpret=False):
    return pl.pallas_call(
        _prep_body,
        grid=(_B // _BB,),
        in_specs=[pl.BlockSpec((_BB, _S, _P), lambda i: (i, 0, 0))],
        out_specs=[
            pl.BlockSpec((_BB, _K), lambda i: (i, 0)),
            pl.BlockSpec((_BB, _K), lambda i: (i, 0)),
        ],
        out_shape=[
            jax.ShapeDtypeStruct((_B, _K), jnp.int32),
            jax.ShapeDtypeStruct((_B, _K), jnp.float32),
        ],
        compiler_params=pltpu.CompilerParams(
            dimension_semantics=("parallel",),
        ),
        interpret=interpret,
    )(pids)


def _sc_gather_body(flat_hbm, table_hbm, out_hbm, emb0_hbm,
                    idxf_v, a0, a1, b0, b1, zidx_v, e0_v,
                    sa0, sa1, sb0, sb1):
    wid = lax.axis_index("s") * 2 + lax.axis_index("c")
    bbase = wid * _BPW
    pltpu.sync_copy(flat_hbm.at[pl.ds(bbase, _BPW)], idxf_v)

    def _g(b, off, n, rows, sem):
        return pltpu.make_async_copy(
            table_hbm.at[idxf_v.at[b, pl.ds(off, n)]], rows, sem)

    def _w(b, off, n, rows):
        pltpu.sync_copy(
            rows, out_hbm.at[pl.ds((bbase + b) * _K + off, n), pl.ds(0, _D)])

    @pl.when(wid == 0)
    def _():
        zidx_v[...] = jnp.zeros((16,), jnp.int32)
        pltpu.async_copy(table_hbm.at[zidx_v], e0_v, sa0).wait()
        pltpu.sync_copy(e0_v, emb0_hbm.at[:, pl.ds(0, _D)])

    _g(0, 0, _C0, a0, sa0).start()
    _g(0, _C0, _C1, a1, sa1).start()

    def body(t, carry):
        e = 2 * t
        o = e + 1
        _g(o, 0, _C0, b0, sb0).start()
        _g(o, _C0, _C1, b1, sb1).start()
        _g(e, 0, _C0, a0, sa0).wait()
        _w(e, 0, _C0, a0)
        _g(e, _C0, _C1, a1, sa1).wait()
        _w(e, _C0, _C1, a1)

        @pl.when(t < _BPW // 2 - 1)
        def _():
            _g(e + 2, 0, _C0, a0, sa0).start()
            _g(e + 2, _C0, _C1, a1, sa1).start()

        _g(o, 0, _C0, b0, sb0).wait()
        _w(o, 0, _C0, b0)
        _g(o, _C0, _C1, b1, sb1).wait()
        _w(o, _C0, _C1, b1)
        return carry

    lax.fori_loop(0, _BPW // 2, body, 0)


@functools.cache
def _sc_gather():
    return pl.kernel(
        _sc_gather_body,
        mesh=plsc.VectorSubcoreMesh(core_axis_name="c", subcore_axis_name="s"),
        out_type=(
            jax.ShapeDtypeStruct((_TOTAL, 128), jnp.float32),
            jax.ShapeDtypeStruct((16, 128), jnp.float32),
        ),
        scratch_types=[
            pltpu.VMEM((_BPW, _K), jnp.int32),
            pltpu.VMEM((_C0, _D), jnp.float32),
            pltpu.VMEM((_C1, _D), jnp.float32),
            pltpu.VMEM((_C0, _D), jnp.float32),
            pltpu.VMEM((_C1, _D), jnp.float32),
            pltpu.VMEM((16,), jnp.int32),
            pltpu.VMEM((16, _D), jnp.float32),
            pltpu.SemaphoreType.DMA,
            pltpu.SemaphoreType.DMA,
            pltpu.SemaphoreType.DMA,
            pltpu.SemaphoreType.DMA,
        ],
        compiler_params=pltpu.CompilerParams(use_tc_tiling_on_sc=False),
    )


def _tc_body(hid_ref, cand_ref, rw_ref, emb0_ref, out_ref):
    e0 = emb0_ref[0:1, 0:_D]
    row = lax.broadcasted_iota(jnp.int32, (_S, _K), 0)
    col = lax.broadcasted_iota(jnp.int32, (_S, _K), 1)
    bigmask = ((col >> 2) == row).astype(jnp.float32)
    jq = lax.broadcasted_iota(jnp.int32, (_K, _P), 0)
    pq = lax.broadcasted_iota(jnp.int32, (_K, _P), 1)
    emat = ((jq & 3) == pq).astype(jnp.float32)
    for i in range(_BB):
        h = hid_ref[i]
        ce = cand_ref[i * _K : (i + 1) * _K, 0:_D]
        r = rw_ref[i : i + 1]
        padc = _K - jnp.sum(r)
        logits = lax.dot_general(
            h, ce, (((1,), (1,)), ((), ())), preferred_element_type=jnp.float32
        )
        l0 = jnp.sum(h * e0, axis=1, keepdims=True)
        m = jnp.maximum(jnp.max(logits, axis=1, keepdims=True), l0)
        esum = jnp.sum(jnp.exp(logits - m) * r, axis=1, keepdims=True)
        denom = esum + padc * jnp.exp(l0 - m)
        lse = m + jnp.log(denom)
        tsel = lax.dot_general(
            logits * bigmask, emat, (((1,), (0,)), ((), ())),
            preferred_element_type=jnp.float32,
        )
        out_ref[i] = tsel - lse


def _tc_compute(hidden, cand, rw, emb0, interpret=False):
    return pl.pallas_call(
        _tc_body,
        grid=(_B // _BB,),
        in_specs=[
            pl.BlockSpec((_BB, _S, _D), lambda i: (i, 0, 0)),
            pl.BlockSpec((_BB * _K, cand.shape[1]), lambda i: (i, 0)),
            pl.BlockSpec((_BB, _K), lambda i: (i, 0)),
            pl.BlockSpec((16, 128), lambda i: (0, 0)),
        ],
        out_specs=pl.BlockSpec((_BB, _S, _P), lambda i: (i, 0, 0)),
        out_shape=jax.ShapeDtypeStruct((_B, _S, _P), jnp.float32),
        compiler_params=pltpu.CompilerParams(
            dimension_semantics=("parallel",),
        ),
        interpret=interpret,
    )(hidden, cand, rw, emb0)


def kernel(hidden, parent_ids, embedding_table):
    pids = parent_ids.astype(jnp.int32)
    flat, rw = _prep(pids)
    cand, emb0 = _sc_gather()(flat, embedding_table)
    return _tc_compute(hidden, cand, rw, emb0)


# speedup vs baseline: 1.2082x; 1.2082x over previous
import functools

import jax
import jax.numpy as jnp
from jax import lax
from jax.experimental import pallas as pl
from jax.experimental.pallas import tpu as pltpu
from jax.experimental.pallas import tpu_sc as plsc

_NUM_EMB = 1000000
_D = 64
_B, _S, _P = 1024, 50, 4
_K = _S * _P
_TOTAL = _B * _K
_NW = 32
_BPW = _B // _NW
_C0, _C1 = 128, 72
_BB = 16


def _sc_gather_body(flat_hbm, table_hbm, out_hbm, emb0_hbm,
                    idxf_v, a0, a1, b0, b1, zidx_v, e0_v,
                    sa0, sa1, sb0, sb1):
    wid = lax.axis_index("s") * 2 + lax.axis_index("c")
    bbase = wid * _BPW
    pltpu.sync_copy(flat_hbm.at[pl.ds(bbase, _BPW)], idxf_v)

    def _g(b, off, n, rows, sem):
        return pltpu.make_async_copy(
            table_hbm.at[idxf_v.at[b, pl.ds(off, n)]], rows, sem)

    def _w(b, off, n, rows):
        pltpu.sync_copy(
            rows, out_hbm.at[pl.ds((bbase + b) * _K + off, n), pl.ds(0, _D)])

    @pl.when(wid == 0)
    def _():
        zidx_v[...] = jnp.zeros((16,), jnp.int32)
        pltpu.async_copy(table_hbm.at[zidx_v], e0_v, sa0).wait()
        pltpu.sync_copy(e0_v, emb0_hbm.at[:, pl.ds(0, _D)])

    _g(0, 0, _C0, a0, sa0).start()
    _g(0, _C0, _C1, a1, sa1).start()

    def body(t, carry):
        e = 2 * t
        o = e + 1
        _g(o, 0, _C0, b0, sb0).start()
        _g(o, _C0, _C1, b1, sb1).start()
        _g(e, 0, _C0, a0, sa0).wait()
        _w(e, 0, _C0, a0)
        _g(e, _C0, _C1, a1, sa1).wait()
        _w(e, _C0, _C1, a1)

        @pl.when(t < _BPW // 2 - 1)
        def _():
            _g(e + 2, 0, _C0, a0, sa0).start()
            _g(e + 2, _C0, _C1, a1, sa1).start()

        _g(o, 0, _C0, b0, sb0).wait()
        _w(o, 0, _C0, b0)
        _g(o, _C0, _C1, b1, sb1).wait()
        _w(o, _C0, _C1, b1)
        return carry

    lax.fori_loop(0, _BPW // 2, body, 0)


@functools.cache
def _sc_gather():
    return pl.kernel(
        _sc_gather_body,
        mesh=plsc.VectorSubcoreMesh(core_axis_name="c", subcore_axis_name="s"),
        out_type=(
            jax.ShapeDtypeStruct((_TOTAL, 128), jnp.float32),
            jax.ShapeDtypeStruct((16, 128), jnp.float32),
        ),
        scratch_types=[
            pltpu.VMEM((_BPW, _K), jnp.int32),
            pltpu.VMEM((_C0, _D), jnp.float32),
            pltpu.VMEM((_C1, _D), jnp.float32),
            pltpu.VMEM((_C0, _D), jnp.float32),
            pltpu.VMEM((_C1, _D), jnp.float32),
            pltpu.VMEM((16,), jnp.int32),
            pltpu.VMEM((16, _D), jnp.float32),
            pltpu.SemaphoreType.DMA,
            pltpu.SemaphoreType.DMA,
            pltpu.SemaphoreType.DMA,
            pltpu.SemaphoreType.DMA,
        ],
        compiler_params=pltpu.CompilerParams(use_tc_tiling_on_sc=False),
    )


def _tc_body(hid_ref, cand_ref, ids_ref, emb0_ref, out_ref):
    e0 = emb0_ref[0:1, 0:_D]
    row = lax.broadcasted_iota(jnp.int32, (_S, _K), 0)
    col = lax.broadcasted_iota(jnp.int32, (_S, _K), 1)
    bigmask = ((col >> 2) == row).astype(jnp.float32)
    jq = lax.broadcasted_iota(jnp.int32, (_K, _P), 0)
    pq = lax.broadcasted_iota(jnp.int32, (_K, _P), 1)
    emat = ((jq & 3) == pq).astype(jnp.float32)
    kr = lax.broadcasted_iota(jnp.int32, (_K, _K), 0)
    kc = lax.broadcasted_iota(jnp.int32, (_K, _K), 1)
    identf = (kr == kc).astype(jnp.float32)
    for i in range(_BB):
        h = hid_ref[i]
        ce = cand_ref[i * _K : (i + 1) * _K, 0:_D]
        ids_i = ids_ref[i : i + 1]
        cols = []
        for c in ((ids_i & 127), ((ids_i >> 7) & 127), (ids_i >> 14)):
            cols.append(lax.dot_general(
                identf, c.astype(jnp.float32), (((1,), (1,)), ((), ())),
                preferred_element_type=jnp.float32,
            ))
        ids_col = cols[0] + 128.0 * cols[1] + 16384.0 * cols[2]
        ids_row = ids_i.astype(jnp.float32)
        eqf = (ids_col == ids_row).astype(jnp.float32)
        cnt = jnp.sum(eqf, axis=0, keepdims=True)
        r = 1.0 / cnt
        padc = _K - jnp.sum(r)
        logits = lax.dot_general(
            h, ce, (((1,), (1,)), ((), ())), preferred_element_type=jnp.float32
        )
        l0 = jnp.sum(h * e0, axis=1, keepdims=True)
        m = jnp.maximum(jnp.max(logits, axis=1, keepdims=True), l0)
        esum = jnp.sum(jnp.exp(logits - m) * r, axis=1, keepdims=True)
        denom = esum + padc * jnp.exp(l0 - m)
        lse = m + jnp.log(denom)
        tsel = lax.dot_general(
            logits * bigmask, emat, (((1,), (0,)), ((), ())),
            preferred_element_type=jnp.float32,
        )
        out_ref[i] = tsel - lse


def _tc_compute(hidden, cand, flat, emb0, interpret=False):
    return pl.pallas_call(
        _tc_body,
        grid=(_B // _BB,),
        in_specs=[
            pl.BlockSpec((_BB, _S, _D), lambda i: (i, 0, 0)),
            pl.BlockSpec((_BB * _K, cand.shape[1]), lambda i: (i, 0)),
            pl.BlockSpec((_BB, _K), lambda i: (i, 0)),
            pl.BlockSpec((16, 128), lambda i: (0, 0)),
        ],
        out_specs=pl.BlockSpec((_BB, _S, _P), lambda i: (i, 0, 0)),
        out_shape=jax.ShapeDtypeStruct((_B, _S, _P), jnp.float32),
        compiler_params=pltpu.CompilerParams(
            dimension_semantics=("parallel",),
        ),
        interpret=interpret,
    )(hidden, cand, flat, emb0)


def kernel(hidden, parent_ids, embedding_table):
    flat = parent_ids.reshape(_B, _K).astype(jnp.int32)
    cand, emb0 = _sc_gather()(flat, embedding_table)
    return _tc_compute(hidden, cand, flat, emb0)
